# baseline (speedup 1.0000x reference)
import functools

import jax
import jax.numpy as jnp
from jax import lax
from jax.experimental import pallas as pl
from jax.experimental.pallas import tpu as pltpu
from jax.experimental.pallas import tpu_sc as plsc

f32 = jnp.float32
i32 = jnp.int32

N_NODES = 10000
NT = 10240
P = 320000
NW = 32
C = 128
NCH = 80
PW = C * NCH
PPAD = PW * NW
NTA = 10064
RPTA = NTA // 16
RPT = NT // 16


def _vgather(vec, idx):
    dn = lax.GatherDimensionNumbers(
        offset_dims=(), collapsed_slice_dims=(0,), start_index_map=(0,))
    return lax.gather(vec, idx[:, None], dn, slice_sizes=(1,),
                      mode=lax.GatherScatterMode.PROMISE_IN_BOUNDS)


def _tc1_body(x_ref, w1_ref, ahd_ref, amd_ref, atl_ref, hd_o, md_o, tl_o):
    h1 = jnp.dot(x_ref[...], w1_ref[...], preferred_element_type=f32)
    hd_o[...] = jnp.dot(h1, ahd_ref[...], preferred_element_type=f32)
    md_o[...] = jnp.dot(h1, amd_ref[...], preferred_element_type=f32)
    tl_o[:, 0:64] = h1
    tl_o[:, 64:80] = jnp.dot(h1, atl_ref[...], preferred_element_type=f32)


def _sc1_body(tblhd_h, tblmd_h, tbltl_h, hd_h, md_h, tl_h, out_h,
              hd_a, md_a, tl_a, rhd0, rmd0, rtl0, rhd1, rmd1, rtl1,
              ov0, ov1, dummy_i, acc,
              semg0, semg1, sems0, sems1):
    c = lax.axis_index("c")
    s = lax.axis_index("s")
    wid = c * 16 + s
    ib = wid * NCH

    pltpu.sync_copy(hd_h.at[pl.ds(ib, NCH)], hd_a)
    pltpu.sync_copy(md_h.at[pl.ds(ib, NCH)], md_a)
    pltpu.sync_copy(tl_h.at[pl.ds(ib, NCH)], tl_a)
    g0 = (pltpu.async_copy(tblhd_h.at[hd_a.at[0]], rhd0, semg0),
          pltpu.async_copy(tblmd_h.at[md_a.at[0]], rmd0, semg0),
          pltpu.async_copy(tbltl_h.at[tl_a.at[0]], rtl0, semg0))
    del g0

    zero16 = jnp.zeros((16,), f32)
    full_n = jnp.full((16,), N_NODES, i32)

    def _zrow(i, carry):
        for j in range(5):
            ov0[i, pl.ds(16 * j, 16)] = zero16
        return carry

    lax.fori_loop(0, C, _zrow, 0)
    for j in range(8):
        dummy_i[pl.ds(16 * j, 16)] = full_n
    for k in range(4):
        pltpu.sync_copy(ov0, acc.at[pl.ds(s * RPTA + k * C, C)])
    pltpu.sync_copy(ov0.at[pl.ds(0, RPTA - 4 * C)],
                    acc.at[pl.ds(s * RPTA + 4 * C, RPTA - 4 * C)])
    plsc.subcore_barrier()
    pltpu.async_copy(ov0, acc.at[dummy_i], sems0, add=True)
    pltpu.async_copy(ov0, acc.at[dummy_i], sems1, add=True)

    iota = lax.iota(i32, 16)
    upsel = (iota >= 8).astype(i32)

    def _compute(rhd, rmd, rtl, ov):
        def _path(p):
            a = rhd[p, :]
            b = rmd[p, :]
            st = rtl[p, pl.ds(64, 16)]
            tt = a + b + st
            tt = jnp.maximum(tt, 0.2 * tt)
            ex = jnp.exp(tt)
            ov[p, pl.ds(64, 16)] = ex
            for j in range(4):
                m = _vgather(ex, 2 * j + upsel)
                ov[p, pl.ds(16 * j, 16)] = m * rtl[p, pl.ds(16 * j, 16)]
        plsc.parallel_loop(0, C, 1, unroll=4)(_path)

    NU = NCH // 2

    def _pair(u, carry):
        t0 = 2 * u
        t1 = t0 + 1
        pltpu.async_copy(tblhd_h.at[hd_a.at[t1]], rhd1, semg1)
        pltpu.async_copy(tblmd_h.at[md_a.at[t1]], rmd1, semg1)
        pltpu.async_copy(tbltl_h.at[tl_a.at[t1]], rtl1, semg1)
        pltpu.make_async_copy(tblhd_h.at[hd_a.at[t0]], rhd0, semg0).wait()
        pltpu.make_async_copy(tblmd_h.at[md_a.at[t0]], rmd0, semg0).wait()
        pltpu.make_async_copy(tbltl_h.at[tl_a.at[t0]], rtl0, semg0).wait()
        pltpu.make_async_copy(ov0, acc.at[dummy_i], sems0).wait()
        _compute(rhd0, rmd0, rtl0, ov0)
        pltpu.async_copy(ov0, acc.at[hd_a.at[t0]], sems0, add=True)

        @pl.when(u < NU - 1)
        def _prefetch():
            pltpu.async_copy(tblhd_h.at[hd_a.at[t0 + 2]], rhd0, semg0)
            pltpu.async_copy(tblmd_h.at[md_a.at[t0 + 2]], rmd0, semg0)
            pltpu.async_copy(tbltl_h.at[tl_a.at[t0 + 2]], rtl0, semg0)

        pltpu.make_async_copy(tblhd_h.at[hd_a.at[t1]], rhd1, semg1).wait()
        pltpu.make_async_copy(tblmd_h.at[md_a.at[t1]], rmd1, semg1).wait()
        pltpu.make_async_copy(tbltl_h.at[tl_a.at[t1]], rtl1, semg1).wait()
        pltpu.make_async_copy(ov0, acc.at[dummy_i], sems1).wait()
        _compute(rhd1, rmd1, rtl1, ov1)
        pltpu.async_copy(ov1, acc.at[hd_a.at[t1]], sems1, add=True)
        return carry

    lax.fori_loop(0, NU, _pair, 0)
    pltpu.make_async_copy(ov0, acc.at[dummy_i], sems0).wait()
    pltpu.make_async_copy(ov0, acc.at[dummy_i], sems1).wait()
    plsc.subcore_barrier()
    pltpu.sync_copy(acc.at[pl.ds(s * RPTA, RPTA)],
                    out_h.at[c, pl.ds(s * RPTA, RPTA)])


def _tc2_body(p_ref, k_ref, m2tl_ref, m2s_ref, tl_o, s_o):
    num = p_ref[0, :, 0:64] + p_ref[1, :, 0:64]
    den = p_ref[0, :, 64:72] + p_ref[1, :, 64:72]
    den64 = jnp.dot(den, k_ref[...], preferred_element_type=f32)
    agg = num / (den64 + 1e-16)
    hh = jnp.where(agg > 0, agg, jnp.exp(agg) - 1.0)
    ttl = jnp.dot(hh, m2tl_ref[...], preferred_element_type=f32)
    col = lax.broadcasted_iota(i32, ttl.shape, 1)
    tl_o[...] = jnp.where(col == 7, 1.0, ttl)
    s_o[...] = jnp.dot(hh, m2s_ref[...], preferred_element_type=f32)


def _sc2_body(tbl2tl_h, s2all_h, hd_h, md_h, tl_h, out_h,
              hd_a, md_a, tl_a, s2res, rtl0, rtl1, ov0, ov1,
              zbuf, dummy_i, acc,
              semg0, semg1, sems0, sems1):
    c = lax.axis_index("c")
    s = lax.axis_index("s")
    wid = c * 16 + s
    ib = wid * NCH

    pltpu.sync_copy(hd_h.at[pl.ds(ib, NCH)], hd_a)
    pltpu.sync_copy(md_h.at[pl.ds(ib, NCH)], md_a)
    pltpu.sync_copy(tl_h.at[pl.ds(ib, NCH)], tl_a)
    pltpu.async_copy(tbl2tl_h.at[tl_a.at[0]], rtl0, semg0)
    pltpu.sync_copy(s2all_h, s2res)

    zero16 = jnp.zeros((16,), f32)
    full_n = jnp.full((16,), N_NODES, i32)

    def _zrow(i, carry):
        zbuf[i, pl.ds(0, 16)] = zero16
        return carry

    lax.fori_loop(0, C, _zrow, 0)
    for j in range(8):
        dummy_i[pl.ds(16 * j, 16)] = full_n
    for k in range(5):
        pltpu.sync_copy(zbuf, acc.at[pl.ds(s * RPT + k * C, C)])
    plsc.subcore_barrier()
    pltpu.async_copy(zbuf, acc.at[dummy_i], sems0, add=True)
    pltpu.async_copy(zbuf, acc.at[dummy_i], sems1, add=True)

    one_i = jnp.full((16,), 1, i32)
    two_i = jnp.full((16,), 2, i32)

    def _compute(idx_row, rtl, ov):
        def _grp(g):
            hd16 = idx_row[0][pl.ds(g * 16, 16)]
            md16 = idx_row[1][pl.ds(g * 16, 16)]
            tl16 = idx_row[2][pl.ds(g * 16, 16)]
            a = plsc.load_gather(s2res, [hd16 * 4])
            b = plsc.load_gather(s2res, [md16 * 4 + one_i])
            st = plsc.load_gather(s2res, [tl16 * 4 + two_i])
            tt = a + b + st
            tt = jnp.maximum(tt, 0.2 * tt)
            ex = jnp.exp(tt)
            for k in range(16):
                m = _vgather(ex, jnp.full((16,), k, i32))
                ov[g * 16 + k, :] = m * rtl[g * 16 + k, :]
        plsc.parallel_loop(0, C // 16, 1, unroll=2)(_grp)

    NU = NCH // 2

    def _pair(u, carry):
        t0 = 2 * u
        t1 = t0 + 1
        pltpu.async_copy(tbl2tl_h.at[tl_a.at[t1]], rtl1, semg1)
        pltpu.make_async_copy(tbl2tl_h.at[tl_a.at[t0]], rtl0, semg0).wait()
        pltpu.make_async_copy(ov0, acc.at[dummy_i], sems0).wait()
        _compute((hd_a.at[t0], md_a.at[t0], tl_a.at[t0]), rtl0, ov0)
        pltpu.async_copy(ov0, acc.at[hd_a.at[t0]], sems0, add=True)

        @pl.when(u < NU - 1)
        def _prefetch():
            pltpu.async_copy(tbl2tl_h.at[tl_a.at[t0 + 2]], rtl0, semg0)

        pltpu.make_async_copy(tbl2tl_h.at[tl_a.at[t1]], rtl1, semg1).wait()
        pltpu.make_async_copy(ov0, acc.at[dummy_i], sems1).wait()
        _compute((hd_a.at[t1], md_a.at[t1], tl_a.at[t1]), rtl1, ov1)
        pltpu.async_copy(ov1, acc.at[hd_a.at[t1]], sems1, add=True)
        return carry

    lax.fori_loop(0, NU, _pair, 0)
    pltpu.make_async_copy(ov0, acc.at[dummy_i], sems0).wait()
    pltpu.make_async_copy(ov0, acc.at[dummy_i], sems1).wait()
    plsc.subcore_barrier()
    pltpu.sync_copy(acc.at[pl.ds(s * RPT, RPT)],
                    out_h.at[c, pl.ds(s * RPT, RPT)])


def _tc3_body(p_ref, out_ref):
    num = p_ref[0, :, 0:7] + p_ref[1, :, 0:7]
    den = p_ref[0, :, 7:8] + p_ref[1, :, 7:8]
    h = num / (den + 1e-16)
    m = jnp.max(h, axis=1, keepdims=True)
    z = h - m
    out_ref[...] = z - jnp.log(jnp.sum(jnp.exp(z), axis=1, keepdims=True))


def kernel(x, path_index, W1, att1, W2, att2):
    heads = att1.shape[0]
    hid = att1.shape[1] // 3
    ncls = att2.shape[1] // 3
    D = x.shape[1]

    pi = path_index.astype(i32).reshape(3, NW, P // NW)
    npad = PW - P // NW
    padk = jnp.arange(npad, dtype=i32)
    pad_hd = jnp.broadcast_to(N_NODES + padk % (NTA - N_NODES), (NW, npad))
    pad_mt = jnp.full((NW, npad), N_NODES, i32)
    pi = jnp.concatenate(
        [pi, jnp.stack([pad_hd, pad_mt, pad_mt])], axis=2).reshape(3, PPAD)
    hd_arr = pi[0].reshape(PPAD // C, C)
    md_arr = pi[1].reshape(PPAD // C, C)
    tl_arr = pi[2].reshape(PPAD // C, C)
    x_pad = jnp.zeros((NT, D), f32).at[:N_NODES].set(x)

    I8 = jnp.eye(heads, dtype=f32)

    def blockdiag(attpart):
        M = (attpart[:, :, None] * I8[:, None, :]).reshape(heads * hid, heads)
        return jnp.concatenate([M, jnp.zeros((heads * hid, 16 - heads), f32)], axis=1)

    Ahd = blockdiag(att1[:, 0:hid])
    Amd = blockdiag(att1[:, hid:2 * hid])
    Atl = blockdiag(att1[:, 2 * hid:])
    K = jnp.kron(I8, jnp.ones((1, hid), f32))
    a2 = att2[0]
    M2tl = jnp.concatenate([W2, jnp.zeros((64, 16 - ncls), f32)], axis=1)
    M2s = jnp.stack([W2 @ a2[0:ncls], W2 @ a2[ncls:2 * ncls],
                     W2 @ a2[2 * ncls:], jnp.zeros((64,), f32)], axis=1)

    BLK = 1024
    G = NT // BLK

    tbl_hd, tbl_md, tbl_tl = pl.pallas_call(
        _tc1_body,
        grid=(G,),
        in_specs=[
            pl.BlockSpec((BLK, D), lambda i: (i, 0)),
            pl.BlockSpec((D, 64), lambda i: (0, 0)),
            pl.BlockSpec((64, 16), lambda i: (0, 0)),
            pl.BlockSpec((64, 16), lambda i: (0, 0)),
            pl.BlockSpec((64, 16), lambda i: (0, 0)),
        ],
        out_specs=[
            pl.BlockSpec((BLK, 16), lambda i: (i, 0)),
            pl.BlockSpec((BLK, 16), lambda i: (i, 0)),
            pl.BlockSpec((BLK, 80), lambda i: (i, 0)),
        ],
        out_shape=[
            jax.ShapeDtypeStruct((NT, 16), f32),
            jax.ShapeDtypeStruct((NT, 16), f32),
            jax.ShapeDtypeStruct((NT, 80), f32),
        ],
    )(x_pad, W1, Ahd, Amd, Atl)

    mesh = plsc.VectorSubcoreMesh(core_axis_name="c", subcore_axis_name="s")
    sc_params = pltpu.CompilerParams(
        use_tc_tiling_on_sc=False, needs_layout_passes=False)
    sc1 = functools.partial(
        pl.kernel,
        out_type=jax.ShapeDtypeStruct((2, NT, 80), f32),
        mesh=mesh,
        compiler_params=sc_params,
        scratch_types=[
            pltpu.VMEM((NCH, C), i32),
            pltpu.VMEM((NCH, C), i32),
            pltpu.VMEM((NCH, C), i32),
            pltpu.VMEM((C, 16), f32),
            pltpu.VMEM((C, 16), f32),
            pltpu.VMEM((C, 80), f32),
            pltpu.VMEM((C, 16), f32),
            pltpu.VMEM((C, 16), f32),
            pltpu.VMEM((C, 80), f32),
            pltpu.VMEM((C, 80), f32),
            pltpu.VMEM((C, 80), f32),
            pltpu.VMEM((C,), i32),
            pltpu.VMEM_SHARED((NTA, 80), f32),
            pltpu.SemaphoreType.DMA,
            pltpu.SemaphoreType.DMA,
            pltpu.SemaphoreType.DMA,
            pltpu.SemaphoreType.DMA,
        ],
    )(_sc1_body)
    part1 = sc1(tbl_hd, tbl_md, tbl_tl, hd_arr, md_arr, tl_arr)

    tbl2_tl, s2all = pl.pallas_call(
        _tc2_body,
        grid=(G,),
        in_specs=[
            pl.BlockSpec((2, BLK, 80), lambda i: (0, i, 0)),
            pl.BlockSpec((8, 64), lambda i: (0, 0)),
            pl.BlockSpec((64, 16), lambda i: (0, 0)),
            pl.BlockSpec((64, 4), lambda i: (0, 0)),
        ],
        out_specs=[
            pl.BlockSpec((BLK, 16), lambda i: (i, 0)),
            pl.BlockSpec((BLK, 4), lambda i: (i, 0)),
        ],
        out_shape=[
            jax.ShapeDtypeStruct((NT, 16), f32),
            jax.ShapeDtypeStruct((NT, 4), f32),
        ],
    )(part1, K, M2tl, M2s)

    sc2 = functools.partial(
        pl.kernel,
        out_type=jax.ShapeDtypeStruct((2, NT, 16), f32),
        mesh=mesh,
        compiler_params=sc_params,
        scratch_types=[
            pltpu.VMEM((NCH, C), i32),
            pltpu.VMEM((NCH, C), i32),
            pltpu.VMEM((NCH, C), i32),
            pltpu.VMEM((NT * 4,), f32),
            pltpu.VMEM((C, 16), f32),
            pltpu.VMEM((C, 16), f32),
            pltpu.VMEM((C, 16), f32),
            pltpu.VMEM((C, 16), f32),
            pltpu.VMEM((C, 16), f32),
            pltpu.VMEM((C,), i32),
            pltpu.VMEM_SHARED((NT, 16), f32),
            pltpu.SemaphoreType.DMA,
            pltpu.SemaphoreType.DMA,
            pltpu.SemaphoreType.DMA,
            pltpu.SemaphoreType.DMA,
        ],
    )(_sc2_body)
    part2 = sc2(tbl2_tl, s2all.reshape(NT * 4), hd_arr, md_arr, tl_arr)

    OBLK = 1000
    out = pl.pallas_call(
        _tc3_body,
        grid=(N_NODES // OBLK,),
        in_specs=[pl.BlockSpec((2, OBLK, 16), lambda i: (0, i, 0))],
        out_specs=pl.BlockSpec((OBLK, ncls), lambda i: (i, 0)),
        out_shape=jax.ShapeDtypeStruct((N_NODES, ncls), f32),
    )(part2)
    return out


# speedup vs baseline: 1.2097x; 1.2097x over previous
import functools

import jax
import jax.numpy as jnp
from jax import lax
from jax.experimental import pallas as pl
from jax.experimental.pallas import tpu as pltpu
from jax.experimental.pallas import tpu_sc as plsc

f32 = jnp.float32
i32 = jnp.int32

N_NODES = 10000
NT = 10240
P = 320000
NW = 32
C = 128
NCH = 80
PW = C * NCH
PPAD = PW * NW
NTA = 10064
RPTA = NTA // 16
RPT = NT // 16


def _vgather(vec, idx):
    dn = lax.GatherDimensionNumbers(
        offset_dims=(), collapsed_slice_dims=(0,), start_index_map=(0,))
    return lax.gather(vec, idx[:, None], dn, slice_sizes=(1,),
                      mode=lax.GatherScatterMode.PROMISE_IN_BOUNDS)


def _tc1_body(x_ref, whd_ref, wmd_ref, wtl_ref, hd_o, md_o, tl_o):
    x = x_ref[...]
    hd_o[...] = jnp.dot(x, whd_ref[...], preferred_element_type=f32)
    md_o[...] = jnp.dot(x, wmd_ref[...], preferred_element_type=f32)
    tl_o[...] = jnp.dot(x, wtl_ref[...],
                        preferred_element_type=f32).astype(jnp.bfloat16)


def _sc1_body(tblhd_h, tblmd_h, tbltl_h, hd_h, md_h, tl_h, out_h,
              hd_a, md_a, tl_a, rhd0, rmd0, rtl0, rhd1, rmd1, rtl1,
              ov0, ov1, dummy_i, acc,
              semg0, semg1, sems0, sems1):
    c = lax.axis_index("c")
    s = lax.axis_index("s")
    wid = c * 16 + s
    ib = wid * NCH

    pltpu.sync_copy(hd_h.at[pl.ds(ib, NCH)], hd_a)
    pltpu.sync_copy(md_h.at[pl.ds(ib, NCH)], md_a)
    pltpu.sync_copy(tl_h.at[pl.ds(ib, NCH)], tl_a)
    g0 = (pltpu.async_copy(tblhd_h.at[hd_a.at[0]], rhd0, semg0),
          pltpu.async_copy(tblmd_h.at[md_a.at[0]], rmd0, semg0),
          pltpu.async_copy(tbltl_h.at[tl_a.at[0]], rtl0, semg0))
    del g0

    zero16 = jnp.zeros((16,), f32)
    full_n = jnp.full((16,), N_NODES, i32)

    def _zrow(i, carry):
        for j in range(5):
            ov0[i, pl.ds(16 * j, 16)] = zero16
        return carry

    lax.fori_loop(0, C, _zrow, 0)
    for j in range(8):
        dummy_i[pl.ds(16 * j, 16)] = full_n
    for k in range(4):
        pltpu.sync_copy(ov0, acc.at[pl.ds(s * RPTA + k * C, C)])
    pltpu.sync_copy(ov0.at[pl.ds(0, RPTA - 4 * C)],
                    acc.at[pl.ds(s * RPTA + 4 * C, RPTA - 4 * C)])
    plsc.subcore_barrier()
    pltpu.async_copy(ov0, acc.at[dummy_i], sems0, add=True)
    pltpu.async_copy(ov0, acc.at[dummy_i], sems1, add=True)

    iota = lax.iota(i32, 16)
    upsel = (iota >= 8).astype(i32)

    def _compute(rhd, rmd, rtl, ov):
        def _path(p):
            a = rhd[p, :]
            b = rmd[p, :]
            st, _ = plsc.unpack(rtl[p, pl.ds(64, 32)],
                                format=plsc.PackFormat.INTERLEAVED)
            tt = a + b + st
            tt = jnp.maximum(tt, 0.2 * tt)
            ex = jnp.exp(tt)
            ov[p, pl.ds(64, 16)] = ex
            for k in range(2):
                h_lo, h_hi = plsc.unpack(rtl[p, pl.ds(32 * k, 32)],
                                         format=plsc.PackFormat.INTERLEAVED)
                m0 = _vgather(ex, 4 * k + upsel)
                m1 = _vgather(ex, 4 * k + 2 + upsel)
                ov[p, pl.ds(32 * k, 16)] = m0 * h_lo
                ov[p, pl.ds(32 * k + 16, 16)] = m1 * h_hi
        plsc.parallel_loop(0, C, 1, unroll=4)(_path)

    NU = NCH // 2

    def _pair(u, carry):
        t0 = 2 * u
        t1 = t0 + 1
        pltpu.async_copy(tblhd_h.at[hd_a.at[t1]], rhd1, semg1)
        pltpu.async_copy(tblmd_h.at[md_a.at[t1]], rmd1, semg1)
        pltpu.async_copy(tbltl_h.at[tl_a.at[t1]], rtl1, semg1)
        pltpu.make_async_copy(tblhd_h.at[hd_a.at[t0]], rhd0, semg0).wait()
        pltpu.make_async_copy(tblmd_h.at[md_a.at[t0]], rmd0, semg0).wait()
        pltpu.make_async_copy(tbltl_h.at[tl_a.at[t0]], rtl0, semg0).wait()
        pltpu.make_async_copy(ov0, acc.at[dummy_i], sems0).wait()
        _compute(rhd0, rmd0, rtl0, ov0)
        pltpu.async_copy(ov0, acc.at[hd_a.at[t0]], sems0, add=True)

        @pl.when(u < NU - 1)
        def _prefetch():
            pltpu.async_copy(tblhd_h.at[hd_a.at[t0 + 2]], rhd0, semg0)
            pltpu.async_copy(tblmd_h.at[md_a.at[t0 + 2]], rmd0, semg0)
            pltpu.async_copy(tbltl_h.at[tl_a.at[t0 + 2]], rtl0, semg0)

        pltpu.make_async_copy(tblhd_h.at[hd_a.at[t1]], rhd1, semg1).wait()
        pltpu.make_async_copy(tblmd_h.at[md_a.at[t1]], rmd1, semg1).wait()
        pltpu.make_async_copy(tbltl_h.at[tl_a.at[t1]], rtl1, semg1).wait()
        pltpu.make_async_copy(ov0, acc.at[dummy_i], sems1).wait()
        _compute(rhd1, rmd1, rtl1, ov1)
        pltpu.async_copy(ov1, acc.at[hd_a.at[t1]], sems1, add=True)
        return carry

    lax.fori_loop(0, NU, _pair, 0)
    pltpu.make_async_copy(ov0, acc.at[dummy_i], sems0).wait()
    pltpu.make_async_copy(ov0, acc.at[dummy_i], sems1).wait()
    plsc.subcore_barrier()
    pltpu.sync_copy(acc.at[pl.ds(s * RPTA, RPTA)],
                    out_h.at[c, pl.ds(s * RPTA, RPTA)])


def _tc2_body(p_ref, k_ref, m2tl_ref, m2s_ref, tl_o, s_o):
    num = p_ref[0, :, 0:64] + p_ref[1, :, 0:64]
    den = p_ref[0, :, 64:72] + p_ref[1, :, 64:72]
    den64 = jnp.dot(den, k_ref[...], preferred_element_type=f32)
    agg = num / (den64 + 1e-16)
    hh = jnp.where(agg > 0, agg, jnp.exp(agg) - 1.0)
    ttl = jnp.dot(hh, m2tl_ref[...], preferred_element_type=f32)
    col = lax.broadcasted_iota(i32, ttl.shape, 1)
    tl_o[...] = jnp.where(col == 7, 1.0, ttl)
    s_o[...] = jnp.dot(hh, m2s_ref[...], preferred_element_type=f32)


def _sc2_body(tbl2tl_h, s2all_h, hd_h, md_h, tl_h, out_h,
              hd_a, md_a, tl_a, s2res, rtl0, rtl1, ov0, ov1,
              zbuf, dummy_i, acc,
              semg0, semg1, sems0, sems1):
    c = lax.axis_index("c")
    s = lax.axis_index("s")
    wid = c * 16 + s
    ib = wid * NCH

    pltpu.sync_copy(hd_h.at[pl.ds(ib, NCH)], hd_a)
    pltpu.sync_copy(md_h.at[pl.ds(ib, NCH)], md_a)
    pltpu.sync_copy(tl_h.at[pl.ds(ib, NCH)], tl_a)
    pltpu.async_copy(tbl2tl_h.at[tl_a.at[0]], rtl0, semg0)
    pltpu.sync_copy(s2all_h, s2res)

    zero16 = jnp.zeros((16,), f32)
    full_n = jnp.full((16,), N_NODES, i32)

    def _zrow(i, carry):
        zbuf[i, pl.ds(0, 16)] = zero16
        return carry

    lax.fori_loop(0, C, _zrow, 0)
    for j in range(8):
        dummy_i[pl.ds(16 * j, 16)] = full_n
    for k in range(5):
        pltpu.sync_copy(zbuf, acc.at[pl.ds(s * RPT + k * C, C)])
    plsc.subcore_barrier()
    pltpu.async_copy(zbuf, acc.at[dummy_i], sems0, add=True)
    pltpu.async_copy(zbuf, acc.at[dummy_i], sems1, add=True)

    one_i = jnp.full((16,), 1, i32)
    two_i = jnp.full((16,), 2, i32)

    def _compute(idx_row, rtl, ov):
        def _grp(g):
            hd16 = idx_row[0][pl.ds(g * 16, 16)]
            md16 = idx_row[1][pl.ds(g * 16, 16)]
            tl16 = idx_row[2][pl.ds(g * 16, 16)]
            a = plsc.load_gather(s2res, [hd16 * 4])
            b = plsc.load_gather(s2res, [md16 * 4 + one_i])
            st = plsc.load_gather(s2res, [tl16 * 4 + two_i])
            tt = a + b + st
            tt = jnp.maximum(tt, 0.2 * tt)
            ex = jnp.exp(tt)
            for k in range(16):
                m = _vgather(ex, jnp.full((16,), k, i32))
                ov[g * 16 + k, :] = m * rtl[g * 16 + k, :]
        plsc.parallel_loop(0, C // 16, 1, unroll=2)(_grp)

    NU = NCH // 2

    def _pair(u, carry):
        t0 = 2 * u
        t1 = t0 + 1
        pltpu.async_copy(tbl2tl_h.at[tl_a.at[t1]], rtl1, semg1)
        pltpu.make_async_copy(tbl2tl_h.at[tl_a.at[t0]], rtl0, semg0).wait()
        pltpu.make_async_copy(ov0, acc.at[dummy_i], sems0).wait()
        _compute((hd_a.at[t0], md_a.at[t0], tl_a.at[t0]), rtl0, ov0)
        pltpu.async_copy(ov0, acc.at[hd_a.at[t0]], sems0, add=True)

        @pl.when(u < NU - 1)
        def _prefetch():
            pltpu.async_copy(tbl2tl_h.at[tl_a.at[t0 + 2]], rtl0, semg0)

        pltpu.make_async_copy(tbl2tl_h.at[tl_a.at[t1]], rtl1, semg1).wait()
        pltpu.make_async_copy(ov0, acc.at[dummy_i], sems1).wait()
        _compute((hd_a.at[t1], md_a.at[t1], tl_a.at[t1]), rtl1, ov1)
        pltpu.async_copy(ov1, acc.at[hd_a.at[t1]], sems1, add=True)
        return carry

    lax.fori_loop(0, NU, _pair, 0)
    pltpu.make_async_copy(ov0, acc.at[dummy_i], sems0).wait()
    pltpu.make_async_copy(ov0, acc.at[dummy_i], sems1).wait()
    plsc.subcore_barrier()
    pltpu.sync_copy(acc.at[pl.ds(s * RPT, RPT)],
                    out_h.at[c, pl.ds(s * RPT, RPT)])


def _tc3_body(p_ref, out_ref):
    num = p_ref[0, :, 0:7] + p_ref[1, :, 0:7]
    den = p_ref[0, :, 7:8] + p_ref[1, :, 7:8]
    h = num / (den + 1e-16)
    m = jnp.max(h, axis=1, keepdims=True)
    z = h - m
    out_ref[...] = z - jnp.log(jnp.sum(jnp.exp(z), axis=1, keepdims=True))


def kernel(x, path_index, W1, att1, W2, att2):
    heads = att1.shape[0]
    hid = att1.shape[1] // 3
    ncls = att2.shape[1] // 3
    D = x.shape[1]

    pi = path_index.astype(i32).reshape(3, NW, P // NW)
    npad = PW - P // NW
    padk = jnp.arange(npad, dtype=i32)
    pad_hd = jnp.broadcast_to(N_NODES + padk % (NTA - N_NODES), (NW, npad))
    pad_mt = jnp.full((NW, npad), N_NODES, i32)
    pi = jnp.concatenate(
        [pi, jnp.stack([pad_hd, pad_mt, pad_mt])], axis=2).reshape(3, PPAD)
    hd_arr = pi[0].reshape(PPAD // C, C)
    md_arr = pi[1].reshape(PPAD // C, C)
    tl_arr = pi[2].reshape(PPAD // C, C)

    I8 = jnp.eye(heads, dtype=f32)

    def blockdiag(attpart):
        return (attpart[:, :, None] * I8[:, None, :]).reshape(heads * hid, heads)

    z8 = jnp.zeros((heads * hid, 16 - heads), f32)
    W1HD = W1 @ jnp.concatenate([blockdiag(att1[:, 0:hid]), z8], axis=1)
    W1MD = W1 @ jnp.concatenate([blockdiag(att1[:, hid:2 * hid]), z8], axis=1)
    S96 = jnp.concatenate(
        [W1, W1 @ blockdiag(att1[:, 2 * hid:]), jnp.zeros((D, 24), f32)],
        axis=1)
    perm = []
    for k in range(2):
        for i in range(16):
            perm += [32 * k + i, 32 * k + 16 + i]
    for i in range(16):
        perm += [64 + i if i < 8 else 72, 72]
    W1TL = S96[:, jnp.array(perm, dtype=i32)]
    K = jnp.kron(I8, jnp.ones((1, hid), f32))
    a2 = att2[0]
    M2tl = jnp.concatenate([W2, jnp.zeros((64, 16 - ncls), f32)], axis=1)
    M2s = jnp.stack([W2 @ a2[0:ncls], W2 @ a2[ncls:2 * ncls],
                     W2 @ a2[2 * ncls:], jnp.zeros((64,), f32)], axis=1)

    BLK = 1024
    G = NT // BLK
    XBLK = 2000

    tbl_hd, tbl_md, tbl_tl = pl.pallas_call(
        _tc1_body,
        grid=(N_NODES // XBLK,),
        in_specs=[
            pl.BlockSpec((XBLK, D), lambda i: (i, 0)),
            pl.BlockSpec((D, 16), lambda i: (0, 0)),
            pl.BlockSpec((D, 16), lambda i: (0, 0)),
            pl.BlockSpec((D, 96), lambda i: (0, 0)),
        ],
        out_specs=[
            pl.BlockSpec((XBLK, 16), lambda i: (i, 0)),
            pl.BlockSpec((XBLK, 16), lambda i: (i, 0)),
            pl.BlockSpec((XBLK, 96), lambda i: (i, 0)),
        ],
        out_shape=[
            jax.ShapeDtypeStruct((NT, 16), f32),
            jax.ShapeDtypeStruct((NT, 16), f32),
            jax.ShapeDtypeStruct((NT, 96), jnp.bfloat16),
        ],
    )(x, W1HD, W1MD, W1TL)

    mesh = plsc.VectorSubcoreMesh(core_axis_name="c", subcore_axis_name="s")
    sc_params = pltpu.CompilerParams(
        use_tc_tiling_on_sc=False, needs_layout_passes=False)
    sc1 = functools.partial(
        pl.kernel,
        out_type=jax.ShapeDtypeStruct((2, NT, 80), f32),
        mesh=mesh,
        compiler_params=sc_params,
        scratch_types=[
            pltpu.VMEM((NCH, C), i32),
            pltpu.VMEM((NCH, C), i32),
            pltpu.VMEM((NCH, C), i32),
            pltpu.VMEM((C, 16), f32),
            pltpu.VMEM((C, 16), f32),
            pltpu.VMEM((C, 96), jnp.bfloat16),
            pltpu.VMEM((C, 16), f32),
            pltpu.VMEM((C, 16), f32),
            pltpu.VMEM((C, 96), jnp.bfloat16),
            pltpu.VMEM((C, 80), f32),
            pltpu.VMEM((C, 80), f32),
            pltpu.VMEM((C,), i32),
            pltpu.VMEM_SHARED((NTA, 80), f32),
            pltpu.SemaphoreType.DMA,
            pltpu.SemaphoreType.DMA,
            pltpu.SemaphoreType.DMA,
            pltpu.SemaphoreType.DMA,
        ],
    )(_sc1_body)
    part1 = sc1(tbl_hd, tbl_md, tbl_tl, hd_arr, md_arr, tl_arr)

    tbl2_tl, s2all = pl.pallas_call(
        _tc2_body,
        grid=(G,),
        in_specs=[
            pl.BlockSpec((2, BLK, 80), lambda i: (0, i, 0)),
            pl.BlockSpec((8, 64), lambda i: (0, 0)),
            pl.BlockSpec((64, 16), lambda i: (0, 0)),
            pl.BlockSpec((64, 4), lambda i: (0, 0)),
        ],
        out_specs=[
            pl.BlockSpec((BLK, 16), lambda i: (i, 0)),
            pl.BlockSpec((BLK, 4), lambda i: (i, 0)),
        ],
        out_shape=[
            jax.ShapeDtypeStruct((NT, 16), f32),
            jax.ShapeDtypeStruct((NT, 4), f32),
        ],
    )(part1, K, M2tl, M2s)

    sc2 = functools.partial(
        pl.kernel,
        out_type=jax.ShapeDtypeStruct((2, NT, 16), f32),
        mesh=mesh,
        compiler_params=sc_params,
        scratch_types=[
            pltpu.VMEM((NCH, C), i32),
            pltpu.VMEM((NCH, C), i32),
            pltpu.VMEM((NCH, C), i32),
            pltpu.VMEM((NT * 4,), f32),
            pltpu.VMEM((C, 16), f32),
            pltpu.VMEM((C, 16), f32),
            pltpu.VMEM((C, 16), f32),
            pltpu.VMEM((C, 16), f32),
            pltpu.VMEM((C, 16), f32),
            pltpu.VMEM((C,), i32),
            pltpu.VMEM_SHARED((NT, 16), f32),
            pltpu.SemaphoreType.DMA,
            pltpu.SemaphoreType.DMA,
            pltpu.SemaphoreType.DMA,
            pltpu.SemaphoreType.DMA,
        ],
    )(_sc2_body)
    part2 = sc2(tbl2_tl, s2all.reshape(NT * 4), hd_arr, md_arr, tl_arr)

    OBLK = 1000
    out = pl.pallas_call(
        _tc3_body,
        grid=(N_NODES // OBLK,),
        in_specs=[pl.BlockSpec((2, OBLK, 16), lambda i: (0, i, 0))],
        out_specs=pl.BlockSpec((OBLK, ncls), lambda i: (i, 0)),
        out_shape=jax.ShapeDtypeStruct((N_NODES, ncls), f32),
    )(part2)
    return out


# speedup vs baseline: 1.2128x; 1.0026x over previous
import functools

import jax
import jax.numpy as jnp
from jax import lax
from jax.experimental import pallas as pl
from jax.experimental.pallas import tpu as pltpu
from jax.experimental.pallas import tpu_sc as plsc

f32 = jnp.float32
i32 = jnp.int32

N_NODES = 10000
NT = 10240
P = 320000
NW = 32
C = 128
NCH = 80
PW = C * NCH
PPAD = PW * NW
NTA = 10064
RPTA = NTA // 16
RPT = NT // 16


def _vgather(vec, idx):
    dn = lax.GatherDimensionNumbers(
        offset_dims=(), collapsed_slice_dims=(0,), start_index_map=(0,))
    return lax.gather(vec, idx[:, None], dn, slice_sizes=(1,),
                      mode=lax.GatherScatterMode.PROMISE_IN_BOUNDS)


def _tc1_body(x_ref, whd_ref, wmd_ref, wtl_ref, hd_o, md_o, tl_o):
    x = x_ref[...]
    hd_o[...] = jnp.dot(x, whd_ref[...], preferred_element_type=f32)
    md_o[...] = jnp.dot(x, wmd_ref[...], preferred_element_type=f32)
    tl_o[...] = jnp.dot(x, wtl_ref[...],
                        preferred_element_type=f32).astype(jnp.bfloat16)


def _sc1_body(tblhd_h, tblmd_h, tbltl_h, idx_h, out_h,
              hd_a, md_a, tl_a, rhd0, rmd0, rtl0, rhd1, rmd1, rtl1,
              ov0, ov1, dummy_i, acc,
              semg0, semg1, sems0, sems1):
    c = lax.axis_index("c")
    s = lax.axis_index("s")
    wid = c * 16 + s
    ib = wid * NCH

    pltpu.sync_copy(idx_h.at[0, pl.ds(ib, NCH)], hd_a)
    pltpu.sync_copy(idx_h.at[1, pl.ds(ib, NCH)], md_a)
    pltpu.sync_copy(idx_h.at[2, pl.ds(ib, NCH)], tl_a)
    g0 = (pltpu.async_copy(tblhd_h.at[hd_a.at[0]], rhd0, semg0),
          pltpu.async_copy(tblmd_h.at[md_a.at[0]], rmd0, semg0),
          pltpu.async_copy(tbltl_h.at[tl_a.at[0]], rtl0, semg0))
    del g0

    zero16 = jnp.zeros((16,), f32)
    full_n = jnp.full((16,), N_NODES, i32)

    def _zrow(i, carry):
        for j in range(5):
            ov0[i, pl.ds(16 * j, 16)] = zero16
        return carry

    lax.fori_loop(0, C, _zrow, 0)
    for j in range(8):
        dummy_i[pl.ds(16 * j, 16)] = full_n
    for k in range(4):
        pltpu.sync_copy(ov0, acc.at[pl.ds(s * RPTA + k * C, C)])
    pltpu.sync_copy(ov0.at[pl.ds(0, RPTA - 4 * C)],
                    acc.at[pl.ds(s * RPTA + 4 * C, RPTA - 4 * C)])
    plsc.subcore_barrier()
    pltpu.async_copy(ov0, acc.at[dummy_i], sems0, add=True)
    pltpu.async_copy(ov0, acc.at[dummy_i], sems1, add=True)

    iota = lax.iota(i32, 16)
    upsel = (iota >= 8).astype(i32)

    def _compute(rhd, rmd, rtl, ov):
        def _path(p):
            a = rhd[p, :]
            b = rmd[p, :]
            st, _ = plsc.unpack(rtl[p, pl.ds(64, 32)],
                                format=plsc.PackFormat.INTERLEAVED)
            tt = a + b + st
            tt = jnp.maximum(tt, 0.2 * tt)
            ex = jnp.exp(tt)
            ov[p, pl.ds(64, 16)] = ex
            for k in range(2):
                h_lo, h_hi = plsc.unpack(rtl[p, pl.ds(32 * k, 32)],
                                         format=plsc.PackFormat.INTERLEAVED)
                m0 = _vgather(ex, 4 * k + upsel)
                m1 = _vgather(ex, 4 * k + 2 + upsel)
                ov[p, pl.ds(32 * k, 16)] = m0 * h_lo
                ov[p, pl.ds(32 * k + 16, 16)] = m1 * h_hi
        plsc.parallel_loop(0, C, 1, unroll=4)(_path)

    NU = NCH // 2

    def _pair(u, carry):
        t0 = 2 * u
        t1 = t0 + 1
        pltpu.async_copy(tblhd_h.at[hd_a.at[t1]], rhd1, semg1)
        pltpu.async_copy(tblmd_h.at[md_a.at[t1]], rmd1, semg1)
        pltpu.async_copy(tbltl_h.at[tl_a.at[t1]], rtl1, semg1)
        pltpu.make_async_copy(tblhd_h.at[hd_a.at[t0]], rhd0, semg0).wait()
        pltpu.make_async_copy(tblmd_h.at[md_a.at[t0]], rmd0, semg0).wait()
        pltpu.make_async_copy(tbltl_h.at[tl_a.at[t0]], rtl0, semg0).wait()
        pltpu.make_async_copy(ov0, acc.at[dummy_i], sems0).wait()
        _compute(rhd0, rmd0, rtl0, ov0)
        pltpu.async_copy(ov0, acc.at[hd_a.at[t0]], sems0, add=True)

        @pl.when(u < NU - 1)
        def _prefetch():
            pltpu.async_copy(tblhd_h.at[hd_a.at[t0 + 2]], rhd0, semg0)
            pltpu.async_copy(tblmd_h.at[md_a.at[t0 + 2]], rmd0, semg0)
            pltpu.async_copy(tbltl_h.at[tl_a.at[t0 + 2]], rtl0, semg0)

        pltpu.make_async_copy(tblhd_h.at[hd_a.at[t1]], rhd1, semg1).wait()
        pltpu.make_async_copy(tblmd_h.at[md_a.at[t1]], rmd1, semg1).wait()
        pltpu.make_async_copy(tbltl_h.at[tl_a.at[t1]], rtl1, semg1).wait()
        pltpu.make_async_copy(ov0, acc.at[dummy_i], sems1).wait()
        _compute(rhd1, rmd1, rtl1, ov1)
        pltpu.async_copy(ov1, acc.at[hd_a.at[t1]], sems1, add=True)
        return carry

    lax.fori_loop(0, NU, _pair, 0)
    pltpu.make_async_copy(ov0, acc.at[dummy_i], sems0).wait()
    pltpu.make_async_copy(ov0, acc.at[dummy_i], sems1).wait()
    plsc.subcore_barrier()
    pltpu.sync_copy(acc.at[pl.ds(s * RPTA, RPTA)],
                    out_h.at[c, pl.ds(s * RPTA, RPTA)])


def _tc2_body(p_ref, k_ref, m2tl_ref, m2s_ref, tl_o, s_o):
    num = p_ref[0, :, 0:64] + p_ref[1, :, 0:64]
    den = p_ref[0, :, 64:72] + p_ref[1, :, 64:72]
    den64 = jnp.dot(den, k_ref[...], preferred_element_type=f32)
    agg = num / (den64 + 1e-16)
    hh = jnp.where(agg > 0, agg, jnp.exp(agg) - 1.0)
    ttl = jnp.dot(hh, m2tl_ref[...], preferred_element_type=f32)
    col = lax.broadcasted_iota(i32, ttl.shape, 1)
    tl_o[...] = jnp.where(col == 7, 1.0, ttl)
    s_o[...] = jnp.dot(hh, m2s_ref[...], preferred_element_type=f32)


def _sc2_body(tbl2tl_h, s2all_h, idx_h, out_h,
              hd_a, md_a, tl_a, s2res, rtl0, rtl1, ov0, ov1,
              zbuf, dummy_i, acc,
              semg0, semg1, sems0, sems1):
    c = lax.axis_index("c")
    s = lax.axis_index("s")
    wid = c * 16 + s
    ib = wid * NCH

    pltpu.sync_copy(idx_h.at[0, pl.ds(ib, NCH)], hd_a)
    pltpu.sync_copy(idx_h.at[1, pl.ds(ib, NCH)], md_a)
    pltpu.sync_copy(idx_h.at[2, pl.ds(ib, NCH)], tl_a)
    pltpu.async_copy(tbl2tl_h.at[tl_a.at[0]], rtl0, semg0)
    pltpu.sync_copy(s2all_h, s2res)

    zero16 = jnp.zeros((16,), f32)
    full_n = jnp.full((16,), N_NODES, i32)

    def _zrow(i, carry):
        zbuf[i, pl.ds(0, 16)] = zero16
        return carry

    lax.fori_loop(0, C, _zrow, 0)
    for j in range(8):
        dummy_i[pl.ds(16 * j, 16)] = full_n
    for k in range(5):
        pltpu.sync_copy(zbuf, acc.at[pl.ds(s * RPT + k * C, C)])
    plsc.subcore_barrier()
    pltpu.async_copy(zbuf, acc.at[dummy_i], sems0, add=True)
    pltpu.async_copy(zbuf, acc.at[dummy_i], sems1, add=True)

    one_i = jnp.full((16,), 1, i32)
    two_i = jnp.full((16,), 2, i32)

    def _compute(idx_row, rtl, ov):
        def _grp(g):
            hd16 = idx_row[0][pl.ds(g * 16, 16)]
            md16 = idx_row[1][pl.ds(g * 16, 16)]
            tl16 = idx_row[2][pl.ds(g * 16, 16)]
            a = plsc.load_gather(s2res, [hd16 * 4])
            b = plsc.load_gather(s2res, [md16 * 4 + one_i])
            st = plsc.load_gather(s2res, [tl16 * 4 + two_i])
            tt = a + b + st
            tt = jnp.maximum(tt, 0.2 * tt)
            ex = jnp.exp(tt)
            for k in range(16):
                m = _vgather(ex, jnp.full((16,), k, i32))
                ov[g * 16 + k, :] = m * rtl[g * 16 + k, :]
        plsc.parallel_loop(0, C // 16, 1, unroll=2)(_grp)

    NU = NCH // 2

    def _pair(u, carry):
        t0 = 2 * u
        t1 = t0 + 1
        pltpu.async_copy(tbl2tl_h.at[tl_a.at[t1]], rtl1, semg1)
        pltpu.make_async_copy(tbl2tl_h.at[tl_a.at[t0]], rtl0, semg0).wait()
        pltpu.make_async_copy(ov0, acc.at[dummy_i], sems0).wait()
        _compute((hd_a.at[t0], md_a.at[t0], tl_a.at[t0]), rtl0, ov0)
        pltpu.async_copy(ov0, acc.at[hd_a.at[t0]], sems0, add=True)

        @pl.when(u < NU - 1)
        def _prefetch():
            pltpu.async_copy(tbl2tl_h.at[tl_a.at[t0 + 2]], rtl0, semg0)

        pltpu.make_async_copy(tbl2tl_h.at[tl_a.at[t1]], rtl1, semg1).wait()
        pltpu.make_async_copy(ov0, acc.at[dummy_i], sems1).wait()
        _compute((hd_a.at[t1], md_a.at[t1], tl_a.at[t1]), rtl1, ov1)
        pltpu.async_copy(ov1, acc.at[hd_a.at[t1]], sems1, add=True)
        return carry

    lax.fori_loop(0, NU, _pair, 0)
    pltpu.make_async_copy(ov0, acc.at[dummy_i], sems0).wait()
    pltpu.make_async_copy(ov0, acc.at[dummy_i], sems1).wait()
    plsc.subcore_barrier()
    pltpu.sync_copy(acc.at[pl.ds(s * RPT, RPT)],
                    out_h.at[c, pl.ds(s * RPT, RPT)])


def _tc3_body(p_ref, out_ref):
    num = p_ref[0, :, 0:7] + p_ref[1, :, 0:7]
    den = p_ref[0, :, 7:8] + p_ref[1, :, 7:8]
    h = num / (den + 1e-16)
    m = jnp.max(h, axis=1, keepdims=True)
    z = h - m
    out_ref[...] = z - jnp.log(jnp.sum(jnp.exp(z), axis=1, keepdims=True))


def kernel(x, path_index, W1, att1, W2, att2):
    heads = att1.shape[0]
    hid = att1.shape[1] // 3
    ncls = att2.shape[1] // 3
    D = x.shape[1]

    pi = path_index.astype(i32).reshape(3, NW, P // NW)
    npad = PW - P // NW
    padk = jnp.arange(npad, dtype=i32)
    pad_hd = jnp.broadcast_to(N_NODES + padk % (NTA - N_NODES), (NW, npad))
    pad_mt = jnp.full((NW, npad), N_NODES, i32)
    idx_all = jnp.concatenate(
        [pi, jnp.stack([pad_hd, pad_mt, pad_mt])], axis=2
    ).reshape(3, PPAD // C, C)

    I8 = jnp.eye(heads, dtype=f32)

    def blockdiag(attpart):
        return (attpart[:, :, None] * I8[:, None, :]).reshape(heads * hid, heads)

    z8 = jnp.zeros((heads * hid, 16 - heads), f32)
    W1HD = W1 @ jnp.concatenate([blockdiag(att1[:, 0:hid]), z8], axis=1)
    W1MD = W1 @ jnp.concatenate([blockdiag(att1[:, hid:2 * hid]), z8], axis=1)
    S96 = jnp.concatenate(
        [W1, W1 @ blockdiag(att1[:, 2 * hid:]), jnp.zeros((D, 24), f32)],
        axis=1)
    perm = []
    for k in range(2):
        for i in range(16):
            perm += [32 * k + i, 32 * k + 16 + i]
    for i in range(16):
        perm += [64 + i if i < 8 else 72, 72]
    W1TL = S96[:, jnp.array(perm, dtype=i32)]
    K = jnp.kron(I8, jnp.ones((1, hid), f32))
    a2 = att2[0]
    M2tl = jnp.concatenate([W2, jnp.zeros((64, 16 - ncls), f32)], axis=1)
    M2s = jnp.stack([W2 @ a2[0:ncls], W2 @ a2[ncls:2 * ncls],
                     W2 @ a2[2 * ncls:], jnp.zeros((64,), f32)], axis=1)

    BLK = 1024
    G = NT // BLK
    XBLK = 2000

    tbl_hd, tbl_md, tbl_tl = pl.pallas_call(
        _tc1_body,
        grid=(N_NODES // XBLK,),
        in_specs=[
            pl.BlockSpec((XBLK, D), lambda i: (i, 0)),
            pl.BlockSpec((D, 16), lambda i: (0, 0)),
            pl.BlockSpec((D, 16), lambda i: (0, 0)),
            pl.BlockSpec((D, 96), lambda i: (0, 0)),
        ],
        out_specs=[
            pl.BlockSpec((XBLK, 16), lambda i: (i, 0)),
            pl.BlockSpec((XBLK, 16), lambda i: (i, 0)),
            pl.BlockSpec((XBLK, 96), lambda i: (i, 0)),
        ],
        out_shape=[
            jax.ShapeDtypeStruct((NT, 16), f32),
            jax.ShapeDtypeStruct((NT, 16), f32),
            jax.ShapeDtypeStruct((NT, 96), jnp.bfloat16),
        ],
    )(x, W1HD, W1MD, W1TL)

    mesh = plsc.VectorSubcoreMesh(core_axis_name="c", subcore_axis_name="s")
    sc_params = pltpu.CompilerParams(
        use_tc_tiling_on_sc=False, needs_layout_passes=False)
    sc1 = functools.partial(
        pl.kernel,
        out_type=jax.ShapeDtypeStruct((2, NT, 80), f32),
        mesh=mesh,
        compiler_params=sc_params,
        scratch_types=[
            pltpu.VMEM((NCH, C), i32),
            pltpu.VMEM((NCH, C), i32),
            pltpu.VMEM((NCH, C), i32),
            pltpu.VMEM((C, 16), f32),
            pltpu.VMEM((C, 16), f32),
            pltpu.VMEM((C, 96), jnp.bfloat16),
            pltpu.VMEM((C, 16), f32),
            pltpu.VMEM((C, 16), f32),
            pltpu.VMEM((C, 96), jnp.bfloat16),
            pltpu.VMEM((C, 80), f32),
            pltpu.VMEM((C, 80), f32),
            pltpu.VMEM((C,), i32),
            pltpu.VMEM_SHARED((NTA, 80), f32),
            pltpu.SemaphoreType.DMA,
            pltpu.SemaphoreType.DMA,
            pltpu.SemaphoreType.DMA,
            pltpu.SemaphoreType.DMA,
        ],
    )(_sc1_body)
    part1 = sc1(tbl_hd, tbl_md, tbl_tl, idx_all)

    tbl2_tl, s2all = pl.pallas_call(
        _tc2_body,
        grid=(G,),
        in_specs=[
            pl.BlockSpec((2, BLK, 80), lambda i: (0, i, 0)),
            pl.BlockSpec((8, 64), lambda i: (0, 0)),
            pl.BlockSpec((64, 16), lambda i: (0, 0)),
            pl.BlockSpec((64, 4), lambda i: (0, 0)),
        ],
        out_specs=[
            pl.BlockSpec((BLK, 16), lambda i: (i, 0)),
            pl.BlockSpec((BLK, 4), lambda i: (i, 0)),
        ],
        out_shape=[
            jax.ShapeDtypeStruct((NT, 16), f32),
            jax.ShapeDtypeStruct((NT, 4), f32),
        ],
    )(part1, K, M2tl, M2s)

    sc2 = functools.partial(
        pl.kernel,
        out_type=jax.ShapeDtypeStruct((2, NT, 16), f32),
        mesh=mesh,
        compiler_params=sc_params,
        scratch_types=[
            pltpu.VMEM((NCH, C), i32),
            pltpu.VMEM((NCH, C), i32),
            pltpu.VMEM((NCH, C), i32),
            pltpu.VMEM((NT * 4,), f32),
            pltpu.VMEM((C, 16), f32),
            pltpu.VMEM((C, 16), f32),
            pltpu.VMEM((C, 16), f32),
            pltpu.VMEM((C, 16), f32),
            pltpu.VMEM((C, 16), f32),
            pltpu.VMEM((C,), i32),
            pltpu.VMEM_SHARED((NT, 16), f32),
            pltpu.SemaphoreType.DMA,
            pltpu.SemaphoreType.DMA,
            pltpu.SemaphoreType.DMA,
            pltpu.SemaphoreType.DMA,
        ],
    )(_sc2_body)
    part2 = sc2(tbl2_tl, s2all.reshape(NT * 4), idx_all)

    OBLK = 1000
    out = pl.pallas_call(
        _tc3_body,
        grid=(N_NODES // OBLK,),
        in_specs=[pl.BlockSpec((2, OBLK, 16), lambda i: (0, i, 0))],
        out_specs=pl.BlockSpec((OBLK, ncls), lambda i: (i, 0)),
        out_shape=jax.ShapeDtypeStruct((N_NODES, ncls), f32),
    )(part2)
    return out
